# baseline (speedup 1.0000x reference)
import functools

import jax
import jax.numpy as jnp
from jax import lax
from jax.experimental import pallas as pl
from jax.experimental.pallas import tpu as pltpu
from jax.experimental.pallas import tpu_sc as plsc

_N_NODES = 10000
_N_EDGES = 320000
_H = 128
_EDGE_DIM = 16
_NUM_REL = 32

_NC = 2
_NS = 16
_NW = _NC * _NS
_EDGES_PER_TILE = _N_EDGES // _NW
_CHUNK = 80
_NCHUNK = _EDGES_PER_TILE // _CHUNK
_N_PAD = 10240
_ROWS_PER_TILE = _N_PAD // _NS
_ZROWS = 80
_NZ = _ROWS_PER_TILE // _ZROWS

_NODE_BLK = 2000
_EDGE_BLK = 2000


def _sc_scatter_body(dst_hbm, src_hbm, hc_hbm, p_hbm, conf_hbm, out_hbm,
                     dstv0, dstv1, srcv0, srcv1, hbuf0, hbuf1, pbuf0, pbuf1,
                     cbuf0, cbuf1, aggr_sh, sl0, sl1, sg0, sg1):
    c = lax.axis_index("c")
    s = lax.axis_index("s")
    wid = s * _NC + c
    base0 = wid * _EDGES_PER_TILE
    dstv = (dstv0, dstv1)
    srcv = (srcv0, srcv1)
    hbuf = (hbuf0, hbuf1)
    pbuf = (pbuf0, pbuf1)
    cbuf = (cbuf0, cbuf1)
    sl = (sl0, sl1)
    sg = (sg0, sg1)

    def issue_loads(g, b):
        base = base0 + g * _CHUNK
        pltpu.async_copy(dst_hbm.at[pl.ds(base, _CHUNK)], dstv[b], sl[b])
        pltpu.async_copy(src_hbm.at[pl.ds(base, _CHUNK)], srcv[b], sl[b])
        pltpu.async_copy(hc_hbm.at[pl.ds(base, _CHUNK)], hbuf[b], sl[b])

    def wait_loads(g, b):
        base = base0 + g * _CHUNK
        pltpu.make_async_copy(dst_hbm.at[pl.ds(base, _CHUNK)], dstv[b], sl[b]).wait()
        pltpu.make_async_copy(src_hbm.at[pl.ds(base, _CHUNK)], srcv[b], sl[b]).wait()
        pltpu.make_async_copy(hc_hbm.at[pl.ds(base, _CHUNK)], hbuf[b], sl[b]).wait()

    def issue_gather(b):
        pltpu.async_copy(p_hbm.at[srcv[b]], pbuf[b], sg[b])
        pltpu.async_copy(conf_hbm.at[srcv[b]], cbuf[b], sg[b])

    def wait_gather(b):
        pltpu.make_async_copy(p_hbm.at[srcv[b]], pbuf[b], sg[b]).wait()
        pltpu.make_async_copy(conf_hbm.at[srcv[b]], cbuf[b], sg[b]).wait()

    gdn = lax.GatherDimensionNumbers(
        offset_dims=(), collapsed_slice_dims=(0,), start_index_map=(0,))

    def compute(b):
        hb, pb, cb = hbuf[b], pbuf[b], cbuf[b]

        def group_body(grp, carry):
            ebase = grp * 16
            cvec = cb[pl.ds(ebase, 16)]
            for i in range(16):
                ci = lax.gather(
                    cvec, jnp.full((16, 1), i, jnp.int32), gdn,
                    slice_sizes=(1,),
                    mode=lax.GatherScatterMode.PROMISE_IN_BOUNDS)
                for j in range(_H // 16):
                    cols = pl.ds(j * 16, 16)
                    hb[ebase + i, cols] = hb[ebase + i, cols] * ci + pb[ebase + i, cols]
            return carry

        lax.fori_loop(0, _CHUNK // 16, group_body, 0)

    def scatter(b):
        pltpu.sync_copy(hbuf[b], aggr_sh.at[dstv[b]], add=True)

    issue_loads(0, 0)
    issue_loads(1, 1)

    z16 = jnp.zeros((16,), jnp.float32)

    def zrow(i, carry):
        for j in range(_H // 16):
            pbuf0[i, pl.ds(j * 16, 16)] = z16
        return carry

    lax.fori_loop(0, _ZROWS, zrow, 0)
    for k in range(_NZ):
        pltpu.sync_copy(
            pbuf0, aggr_sh.at[pl.ds(s * _ROWS_PER_TILE + k * _ZROWS, _ZROWS)])
    plsc.subcore_barrier()

    wait_loads(0, 0)
    issue_gather(0)

    def step(g, b):
        wait_loads(g + 1, 1 - b)
        issue_gather(1 - b)
        wait_gather(b)
        compute(b)
        scatter(b)

        @pl.when(g + 2 < _NCHUNK)
        def _():
            issue_loads(g + 2, b)

    def body(k, carry):
        step(2 * k, 0)
        step(2 * k + 1, 1)
        return carry

    lax.fori_loop(0, _NCHUNK // 2, body, 0)
    wait_gather(0)
    compute(0)
    scatter(0)
    plsc.subcore_barrier()

    for k in range(_NZ):
        r = s * _ROWS_PER_TILE + k * _ZROWS
        pltpu.sync_copy(aggr_sh.at[pl.ds(r, _ZROWS)],
                        out_hbm.at[pl.ds(c * _N_PAD + r, _ZROWS)])


def _tc_node_mlp(x_ref, conf_ref, wn_ref, bn_ref, out_ref):
    p = jnp.dot(x_ref[...], wn_ref[...],
                preferred_element_type=jnp.float32) + bn_ref[...]
    out_ref[...] = jnp.maximum(p, 0.0) * conf_ref[...]


def _tc_edge_mlp(ea_ref, et_ref, rel_ref, we1_ref, be1_ref,
                 we2_ref, be2_ref, out_ref):
    we1a = we1_ref[:_EDGE_DIM, :]
    we1b = we1_ref[_EDGE_DIM:, :]
    r1 = jnp.dot(rel_ref[...], we1b, preferred_element_type=jnp.float32)
    oh = (et_ref[...] == lax.broadcasted_iota(jnp.int32, (1, _NUM_REL), 1))
    z = (jnp.dot(ea_ref[...], we1a, preferred_element_type=jnp.float32)
         + jnp.dot(oh.astype(jnp.float32), r1,
                   preferred_element_type=jnp.float32)
         + be1_ref[...])
    z = jnp.maximum(z, 0.0)
    out_ref[...] = jnp.dot(z, we2_ref[...],
                           preferred_element_type=jnp.float32) + be2_ref[...]


def _tc_fuse(x_ref, a0_ref, a1_ref, wg1_ref, wg2_ref, bg_ref,
             gamma_ref, beta_ref, out_ref):
    x = x_ref[...]
    aggr = a0_ref[...] + a1_ref[...]
    gate = jax.nn.sigmoid(
        jnp.dot(x, wg1_ref[...], preferred_element_type=jnp.float32)
        + jnp.dot(aggr, wg2_ref[...], preferred_element_type=jnp.float32)
        + bg_ref[...])
    fused = gate * jnp.tanh(aggr) + (1.0 - gate) * x
    mean = jnp.mean(fused, axis=-1, keepdims=True)
    ctr = fused - mean
    var = jnp.mean(ctr * ctr, axis=-1, keepdims=True)
    out_ref[...] = ctr * lax.rsqrt(var + 1e-5) * gamma_ref[...] + beta_ref[...]


def kernel(x, edge_index, edge_attr, edge_type, confidence, rel_table,
           We1, be1, We2, be2, Wn, bn, Wg, bg, gamma, beta):
    src = edge_index[0]
    dst = edge_index[1]
    conf1 = confidence.reshape(_N_NODES)

    mesh = plsc.VectorSubcoreMesh(core_axis_name="c", subcore_axis_name="s")

    nblk = _N_NODES // _NODE_BLK
    p_nodes = pl.pallas_call(
        _tc_node_mlp,
        grid=(nblk,),
        in_specs=[
            pl.BlockSpec((_NODE_BLK, _H), lambda i: (i, 0)),
            pl.BlockSpec((_NODE_BLK, 1), lambda i: (i, 0)),
            pl.BlockSpec((_H, _H), lambda i: (0, 0)),
            pl.BlockSpec((1, _H), lambda i: (0, 0)),
        ],
        out_specs=pl.BlockSpec((_NODE_BLK, _H), lambda i: (i, 0)),
        out_shape=jax.ShapeDtypeStruct((_N_NODES, _H), jnp.float32),
    )(x, confidence, Wn, bn.reshape(1, _H))

    eblk = _N_EDGES // _EDGE_BLK
    hc = pl.pallas_call(
        _tc_edge_mlp,
        grid=(eblk,),
        in_specs=[
            pl.BlockSpec((_EDGE_BLK, _EDGE_DIM), lambda i: (i, 0)),
            pl.BlockSpec((_EDGE_BLK, 1), lambda i: (i, 0)),
            pl.BlockSpec((_NUM_REL, _EDGE_DIM), lambda i: (0, 0)),
            pl.BlockSpec((_EDGE_DIM + _EDGE_DIM, _H), lambda i: (0, 0)),
            pl.BlockSpec((1, _H), lambda i: (0, 0)),
            pl.BlockSpec((_H, _H), lambda i: (0, 0)),
            pl.BlockSpec((1, _H), lambda i: (0, 0)),
        ],
        out_specs=pl.BlockSpec((_EDGE_BLK, _H), lambda i: (i, 0)),
        out_shape=jax.ShapeDtypeStruct((_N_EDGES, _H), jnp.float32),
    )(edge_attr, edge_type.reshape(_N_EDGES, 1),
      rel_table, We1, be1.reshape(1, _H), We2, be2.reshape(1, _H))

    partials = pl.kernel(
        _sc_scatter_body,
        out_type=jax.ShapeDtypeStruct((_NC * _N_PAD, _H), jnp.float32),
        mesh=mesh,
        scratch_types=[
            pltpu.VMEM((_CHUNK,), jnp.int32),
            pltpu.VMEM((_CHUNK,), jnp.int32),
            pltpu.VMEM((_CHUNK,), jnp.int32),
            pltpu.VMEM((_CHUNK,), jnp.int32),
            pltpu.VMEM((_CHUNK, _H), jnp.float32),
            pltpu.VMEM((_CHUNK, _H), jnp.float32),
            pltpu.VMEM((_CHUNK, _H), jnp.float32),
            pltpu.VMEM((_CHUNK, _H), jnp.float32),
            pltpu.VMEM((_CHUNK,), jnp.float32),
            pltpu.VMEM((_CHUNK,), jnp.float32),
            pltpu.VMEM_SHARED((_N_PAD, _H), jnp.float32),
            pltpu.SemaphoreType.DMA,
            pltpu.SemaphoreType.DMA,
            pltpu.SemaphoreType.DMA,
            pltpu.SemaphoreType.DMA,
        ],
    )(dst, src, hc, p_nodes, conf1)

    a0 = partials[:_N_NODES]
    a1 = partials[_N_PAD:_N_PAD + _N_NODES]

    out = pl.pallas_call(
        _tc_fuse,
        grid=(nblk,),
        in_specs=[
            pl.BlockSpec((_NODE_BLK, _H), lambda i: (i, 0)),
            pl.BlockSpec((_NODE_BLK, _H), lambda i: (i, 0)),
            pl.BlockSpec((_NODE_BLK, _H), lambda i: (i, 0)),
            pl.BlockSpec((_H, _H), lambda i: (0, 0)),
            pl.BlockSpec((_H, _H), lambda i: (0, 0)),
            pl.BlockSpec((1, _H), lambda i: (0, 0)),
            pl.BlockSpec((1, _H), lambda i: (0, 0)),
            pl.BlockSpec((1, _H), lambda i: (0, 0)),
        ],
        out_specs=pl.BlockSpec((_NODE_BLK, _H), lambda i: (i, 0)),
        out_shape=jax.ShapeDtypeStruct((_N_NODES, _H), jnp.float32),
    )(x, a0, a1, Wg[:_H], Wg[_H:], bg.reshape(1, _H),
      gamma.reshape(1, _H), beta.reshape(1, _H))
    return out


# speedup vs baseline: 1.0975x; 1.0975x over previous
import functools

import jax
import jax.numpy as jnp
from jax import lax
from jax.experimental import pallas as pl
from jax.experimental.pallas import tpu as pltpu
from jax.experimental.pallas import tpu_sc as plsc

_N_NODES = 10000
_N_EDGES = 320000
_H = 128
_EDGE_DIM = 16
_NUM_REL = 32

_NC = 2
_NS = 16
_NW = _NC * _NS
_EDGES_PER_TILE = _N_EDGES // _NW
_CHUNK = 80
_NCHUNK = _EDGES_PER_TILE // _CHUNK
_N_PAD = 10240
_ROWS_PER_TILE = _N_PAD // _NS
_ZROWS = 80
_NZ = _ROWS_PER_TILE // _ZROWS

_NODE_BLK = 2000
_EDGE_BLK = 2000


def _sc_gather_conf_body(src_hbm, conf_hbm, out_hbm,
                         idx0, idx1, idx2, idx3, val0, val1, val2, val3,
                         sg0, sg1, sg2, sg3):
    c = lax.axis_index("c")
    s = lax.axis_index("s")
    wid = s * _NC + c
    base0 = wid * _EDGES_PER_TILE
    idx = (idx0, idx1, idx2, idx3)
    val = (val0, val1, val2, val3)
    sg = (sg0, sg1, sg2, sg3)

    def stage_load(g, b):
        base = base0 + g * _CHUNK
        pltpu.sync_copy(src_hbm.at[pl.ds(base, _CHUNK)], idx[b])
        pltpu.async_copy(conf_hbm.at[idx[b]], val[b], sg[b])

    def stage_drain(g, b):
        pltpu.make_async_copy(conf_hbm.at[idx[b]], val[b], sg[b]).wait()
        base = base0 + g * _CHUNK
        pltpu.sync_copy(val[b], out_hbm.at[pl.ds(base, _CHUNK)])

    for g in range(3):
        stage_load(g, g)

    def body(k, carry):
        for r in range(4):
            g = 4 * k + r
            stage_drain(g, r)

            @pl.when(g + 3 < _NCHUNK)
            def _():
                stage_load(g + 3, (r + 3) % 4)
        return carry

    lax.fori_loop(0, (_NCHUNK - 1) // 4, body, 0)
    stage_drain(_NCHUNK - 1, (_NCHUNK - 1) % 4)


def _sc_scatter_body(dst_hbm, src_hbm, hc_hbm, p_hbm, out_hbm,
                     dstv0, dstv1, srcv0, srcv1, hbuf0, hbuf1, pbuf0, pbuf1,
                     aggr_sh, sl0, sl1, sg0, sg1):
    c = lax.axis_index("c")
    s = lax.axis_index("s")
    wid = s * _NC + c
    base0 = wid * _EDGES_PER_TILE
    dstv = (dstv0, dstv1)
    srcv = (srcv0, srcv1)
    hbuf = (hbuf0, hbuf1)
    pbuf = (pbuf0, pbuf1)
    sl = (sl0, sl1)
    sg = (sg0, sg1)

    def issue_loads(g, b):
        base = base0 + g * _CHUNK
        pltpu.async_copy(dst_hbm.at[pl.ds(base, _CHUNK)], dstv[b], sl[b])
        pltpu.async_copy(src_hbm.at[pl.ds(base, _CHUNK)], srcv[b], sl[b])
        pltpu.async_copy(hc_hbm.at[pl.ds(base, _CHUNK)], hbuf[b], sl[b])

    def wait_loads(g, b):
        base = base0 + g * _CHUNK
        pltpu.make_async_copy(dst_hbm.at[pl.ds(base, _CHUNK)], dstv[b], sl[b]).wait()
        pltpu.make_async_copy(src_hbm.at[pl.ds(base, _CHUNK)], srcv[b], sl[b]).wait()
        pltpu.make_async_copy(hc_hbm.at[pl.ds(base, _CHUNK)], hbuf[b], sl[b]).wait()

    def issue_gather(b):
        pltpu.async_copy(p_hbm.at[srcv[b]], pbuf[b], sg[b])

    def wait_gather(b):
        pltpu.make_async_copy(p_hbm.at[srcv[b]], pbuf[b], sg[b]).wait()

    def scatter(b):
        pltpu.sync_copy(hbuf[b], aggr_sh.at[dstv[b]], add=True)
        pltpu.sync_copy(pbuf[b], aggr_sh.at[dstv[b]], add=True)

    issue_loads(0, 0)
    issue_loads(1, 1)

    z16 = jnp.zeros((16,), jnp.float32)

    def zrow(i, carry):
        for j in range(_H // 16):
            pbuf0[i, pl.ds(j * 16, 16)] = z16
        return carry

    lax.fori_loop(0, _ZROWS, zrow, 0)
    for k in range(_NZ):
        pltpu.sync_copy(
            pbuf0, aggr_sh.at[pl.ds(s * _ROWS_PER_TILE + k * _ZROWS, _ZROWS)])
    plsc.subcore_barrier()

    wait_loads(0, 0)
    issue_gather(0)

    def step(g, b):
        wait_loads(g + 1, 1 - b)
        issue_gather(1 - b)
        wait_gather(b)
        scatter(b)

        @pl.when(g + 2 < _NCHUNK)
        def _():
            issue_loads(g + 2, b)

    def body(k, carry):
        step(2 * k, 0)
        step(2 * k + 1, 1)
        return carry

    lax.fori_loop(0, _NCHUNK // 2, body, 0)
    wait_gather(0)
    scatter(0)
    plsc.subcore_barrier()

    for k in range(_NZ):
        r = s * _ROWS_PER_TILE + k * _ZROWS
        pltpu.sync_copy(aggr_sh.at[pl.ds(r, _ZROWS)],
                        out_hbm.at[pl.ds(c * _N_PAD + r, _ZROWS)])


def _tc_edge_node_mlp(ea_ref, et_ref, cs_ref, x_ref, conf_ref, rel_ref,
                      we1_ref, be1_ref, we2_ref, be2_ref, wn_ref, bn_ref,
                      hc_ref, p_ref):
    we1a = we1_ref[:_EDGE_DIM, :]
    we1b = we1_ref[_EDGE_DIM:, :]
    r1 = jnp.dot(rel_ref[...], we1b, preferred_element_type=jnp.float32)
    oh = (et_ref[...] == lax.broadcasted_iota(jnp.int32, (1, _NUM_REL), 1))
    z = (jnp.dot(ea_ref[...], we1a, preferred_element_type=jnp.float32)
         + jnp.dot(oh.astype(jnp.float32), r1,
                   preferred_element_type=jnp.float32)
         + be1_ref[...])
    z = jnp.maximum(z, 0.0)
    h = jnp.dot(z, we2_ref[...], preferred_element_type=jnp.float32) + be2_ref[...]
    hc_ref[...] = h * cs_ref[...]

    @pl.when(pl.program_id(0) < _N_NODES // _NODE_BLK)
    def _():
        p = jnp.dot(x_ref[...], wn_ref[...],
                    preferred_element_type=jnp.float32) + bn_ref[...]
        p_ref[...] = jnp.maximum(p, 0.0) * conf_ref[...]


def _tc_fuse(x_ref, a0_ref, a1_ref, wg1_ref, wg2_ref, bg_ref,
             gamma_ref, beta_ref, out_ref):
    x = x_ref[...]
    aggr = a0_ref[...] + a1_ref[...]
    gate = jax.nn.sigmoid(
        jnp.dot(x, wg1_ref[...], preferred_element_type=jnp.float32)
        + jnp.dot(aggr, wg2_ref[...], preferred_element_type=jnp.float32)
        + bg_ref[...])
    fused = gate * jnp.tanh(aggr) + (1.0 - gate) * x
    mean = jnp.mean(fused, axis=-1, keepdims=True)
    ctr = fused - mean
    var = jnp.mean(ctr * ctr, axis=-1, keepdims=True)
    out_ref[...] = ctr * lax.rsqrt(var + 1e-5) * gamma_ref[...] + beta_ref[...]


def kernel(x, edge_index, edge_attr, edge_type, confidence, rel_table,
           We1, be1, We2, be2, Wn, bn, Wg, bg, gamma, beta):
    src = edge_index[0]
    dst = edge_index[1]
    conf1 = confidence.reshape(_N_NODES)

    mesh = plsc.VectorSubcoreMesh(core_axis_name="c", subcore_axis_name="s")

    csrc = pl.kernel(
        _sc_gather_conf_body,
        out_type=jax.ShapeDtypeStruct((_N_EDGES,), jnp.float32),
        mesh=mesh,
        scratch_types=(
            [pltpu.VMEM((_CHUNK,), jnp.int32)] * 4
            + [pltpu.VMEM((_CHUNK,), jnp.float32)] * 4
            + [pltpu.SemaphoreType.DMA] * 4
        ),
    )(src, conf1)

    nblk = _N_NODES // _NODE_BLK
    eblk = _N_EDGES // _EDGE_BLK
    nlast = nblk - 1
    hc, p_nodes = pl.pallas_call(
        _tc_edge_node_mlp,
        grid=(eblk,),
        in_specs=[
            pl.BlockSpec((_EDGE_BLK, _EDGE_DIM), lambda i: (i, 0)),
            pl.BlockSpec((_EDGE_BLK, 1), lambda i: (i, 0)),
            pl.BlockSpec((_EDGE_BLK, 1), lambda i: (i, 0)),
            pl.BlockSpec((_NODE_BLK, _H), lambda i: (jnp.minimum(i, nlast), 0)),
            pl.BlockSpec((_NODE_BLK, 1), lambda i: (jnp.minimum(i, nlast), 0)),
            pl.BlockSpec((_NUM_REL, _EDGE_DIM), lambda i: (0, 0)),
            pl.BlockSpec((_EDGE_DIM + _EDGE_DIM, _H), lambda i: (0, 0)),
            pl.BlockSpec((1, _H), lambda i: (0, 0)),
            pl.BlockSpec((_H, _H), lambda i: (0, 0)),
            pl.BlockSpec((1, _H), lambda i: (0, 0)),
            pl.BlockSpec((_H, _H), lambda i: (0, 0)),
            pl.BlockSpec((1, _H), lambda i: (0, 0)),
        ],
        out_specs=[
            pl.BlockSpec((_EDGE_BLK, _H), lambda i: (i, 0)),
            pl.BlockSpec((_NODE_BLK, _H), lambda i: (jnp.minimum(i, nlast), 0)),
        ],
        out_shape=[
            jax.ShapeDtypeStruct((_N_EDGES, _H), jnp.float32),
            jax.ShapeDtypeStruct((_N_NODES, _H), jnp.float32),
        ],
    )(edge_attr, edge_type.reshape(_N_EDGES, 1), csrc.reshape(_N_EDGES, 1),
      x, confidence, rel_table, We1, be1.reshape(1, _H), We2,
      be2.reshape(1, _H), Wn, bn.reshape(1, _H))

    partials = pl.kernel(
        _sc_scatter_body,
        out_type=jax.ShapeDtypeStruct((_NC * _N_PAD, _H), jnp.float32),
        mesh=mesh,
        scratch_types=[
            pltpu.VMEM((_CHUNK,), jnp.int32),
            pltpu.VMEM((_CHUNK,), jnp.int32),
            pltpu.VMEM((_CHUNK,), jnp.int32),
            pltpu.VMEM((_CHUNK,), jnp.int32),
            pltpu.VMEM((_CHUNK, _H), jnp.float32),
            pltpu.VMEM((_CHUNK, _H), jnp.float32),
            pltpu.VMEM((_CHUNK, _H), jnp.float32),
            pltpu.VMEM((_CHUNK, _H), jnp.float32),
            pltpu.VMEM_SHARED((_N_PAD, _H), jnp.float32),
            pltpu.SemaphoreType.DMA,
            pltpu.SemaphoreType.DMA,
            pltpu.SemaphoreType.DMA,
            pltpu.SemaphoreType.DMA,
        ],
    )(dst, src, hc, p_nodes)

    a0 = partials[:_N_NODES]
    a1 = partials[_N_PAD:_N_PAD + _N_NODES]

    out = pl.pallas_call(
        _tc_fuse,
        grid=(nblk,),
        in_specs=[
            pl.BlockSpec((_NODE_BLK, _H), lambda i: (i, 0)),
            pl.BlockSpec((_NODE_BLK, _H), lambda i: (i, 0)),
            pl.BlockSpec((_NODE_BLK, _H), lambda i: (i, 0)),
            pl.BlockSpec((_H, _H), lambda i: (0, 0)),
            pl.BlockSpec((_H, _H), lambda i: (0, 0)),
            pl.BlockSpec((1, _H), lambda i: (0, 0)),
            pl.BlockSpec((1, _H), lambda i: (0, 0)),
            pl.BlockSpec((1, _H), lambda i: (0, 0)),
        ],
        out_specs=pl.BlockSpec((_NODE_BLK, _H), lambda i: (i, 0)),
        out_shape=jax.ShapeDtypeStruct((_N_NODES, _H), jnp.float32),
    )(x, a0, a1, Wg[:_H], Wg[_H:], bg.reshape(1, _H),
      gamma.reshape(1, _H), beta.reshape(1, _H))
    return out


# speedup vs baseline: 1.5811x; 1.4406x over previous
import functools

import jax
import jax.numpy as jnp
from jax import lax
from jax.experimental import pallas as pl
from jax.experimental.pallas import tpu as pltpu
from jax.experimental.pallas import tpu_sc as plsc

_N_NODES = 10000
_N_EDGES = 320000
_H = 128
_EDGE_DIM = 16
_NUM_REL = 32

_NC = 2
_NS = 16
_NW = _NC * _NS
_EDGES_PER_TILE = _N_EDGES // _NW
_CHUNK = 80
_NCHUNK = _EDGES_PER_TILE // _CHUNK
_N_PAD = 10240
_ROWS_PER_TILE = _N_PAD // _NS
_ZROWS = 80
_NZ = _ROWS_PER_TILE // _ZROWS

_NODE_BLK = 2000
_EDGE_BLK = 2560


def _sc_gather_conf_body(src_hbm, conf_hbm, out_hbm,
                         idx0, idx1, idx2, idx3, val0, val1, val2, val3,
                         sg0, sg1, sg2, sg3):
    c = lax.axis_index("c")
    s = lax.axis_index("s")
    wid = s * _NC + c
    base0 = wid * _EDGES_PER_TILE
    idx = (idx0, idx1, idx2, idx3)
    val = (val0, val1, val2, val3)
    sg = (sg0, sg1, sg2, sg3)

    def stage_load(g, b):
        base = base0 + g * _CHUNK
        pltpu.sync_copy(src_hbm.at[pl.ds(base, _CHUNK)], idx[b])
        pltpu.async_copy(conf_hbm.at[idx[b]], val[b], sg[b])

    def stage_drain(g, b):
        pltpu.make_async_copy(conf_hbm.at[idx[b]], val[b], sg[b]).wait()
        base = base0 + g * _CHUNK
        pltpu.sync_copy(val[b], out_hbm.at[pl.ds(base, _CHUNK)])

    for g in range(3):
        stage_load(g, g)

    def body(k, carry):
        for r in range(4):
            g = 4 * k + r
            stage_drain(g, r)

            @pl.when(g + 3 < _NCHUNK)
            def _():
                stage_load(g + 3, (r + 3) % 4)
        return carry

    lax.fori_loop(0, (_NCHUNK - 1) // 4, body, 0)
    stage_drain(_NCHUNK - 1, (_NCHUNK - 1) % 4)


def _sc_scatter_body(dst_hbm, src_hbm, hc_hbm, p_hbm, out_hbm,
                     dstv0, dstv1, srcv0, srcv1, hbuf0, hbuf1, pbuf0, pbuf1,
                     aggr_sh, sl0, sl1, sg0, sg1):
    c = lax.axis_index("c")
    s = lax.axis_index("s")
    wid = s * _NC + c
    base0 = wid * _EDGES_PER_TILE
    dstv = (dstv0, dstv1)
    srcv = (srcv0, srcv1)
    hbuf = (hbuf0, hbuf1)
    pbuf = (pbuf0, pbuf1)
    sl = (sl0, sl1)
    sg = (sg0, sg1)

    def issue_loads(g, b):
        base = base0 + g * _CHUNK
        pltpu.async_copy(dst_hbm.at[pl.ds(base, _CHUNK)], dstv[b], sl[b])
        pltpu.async_copy(src_hbm.at[pl.ds(base, _CHUNK)], srcv[b], sl[b])
        pltpu.async_copy(hc_hbm.at[pl.ds(base, _CHUNK)], hbuf[b], sl[b])

    def wait_loads(g, b):
        base = base0 + g * _CHUNK
        pltpu.make_async_copy(dst_hbm.at[pl.ds(base, _CHUNK)], dstv[b], sl[b]).wait()
        pltpu.make_async_copy(src_hbm.at[pl.ds(base, _CHUNK)], srcv[b], sl[b]).wait()
        pltpu.make_async_copy(hc_hbm.at[pl.ds(base, _CHUNK)], hbuf[b], sl[b]).wait()

    def issue_gather(b):
        pltpu.async_copy(p_hbm.at[srcv[b]], pbuf[b], sg[b])

    def wait_gather(b):
        pltpu.make_async_copy(p_hbm.at[srcv[b]], pbuf[b], sg[b]).wait()

    def scatter(b):
        pltpu.sync_copy(hbuf[b], aggr_sh.at[dstv[b]], add=True)
        pltpu.sync_copy(pbuf[b], aggr_sh.at[dstv[b]], add=True)

    issue_loads(0, 0)
    issue_loads(1, 1)

    z16 = jnp.zeros((16,), jnp.float32)

    def zrow(i, carry):
        for j in range(_H // 16):
            pbuf0[i, pl.ds(j * 16, 16)] = z16
        return carry

    lax.fori_loop(0, _ZROWS, zrow, 0)
    for k in range(_NZ):
        pltpu.sync_copy(
            pbuf0, aggr_sh.at[pl.ds(s * _ROWS_PER_TILE + k * _ZROWS, _ZROWS)])
    plsc.subcore_barrier()

    wait_loads(0, 0)
    issue_gather(0)

    def step(g, b):
        wait_loads(g + 1, 1 - b)
        issue_gather(1 - b)
        wait_gather(b)
        scatter(b)

        @pl.when(g + 2 < _NCHUNK)
        def _():
            issue_loads(g + 2, b)

    def body(k, carry):
        step(2 * k, 0)
        step(2 * k + 1, 1)
        return carry

    lax.fori_loop(0, _NCHUNK // 2, body, 0)
    wait_gather(0)
    scatter(0)
    plsc.subcore_barrier()

    for k in range(_NZ):
        r = s * _ROWS_PER_TILE + k * _ZROWS
        pltpu.sync_copy(aggr_sh.at[pl.ds(r, _ZROWS)],
                        out_hbm.at[pl.ds(c * _N_PAD + r, _ZROWS)])


def _tc_edge_node_mlp(ea_ref, et_ref, cs_ref, x_ref, conf_ref, rel_ref,
                      we1_ref, be1_ref, we2_ref, be2_ref, wn_ref, bn_ref,
                      hc_ref, p_ref):
    we1a = we1_ref[:_EDGE_DIM, :]
    we1b = we1_ref[_EDGE_DIM:, :]
    r1 = jnp.dot(rel_ref[...], we1b, preferred_element_type=jnp.float32)
    oh_t = (et_ref[...] == lax.broadcasted_iota(jnp.int32, (_NUM_REL, 1), 0))
    z = (jnp.dot(ea_ref[...], we1a, preferred_element_type=jnp.float32)
         + lax.dot_general(oh_t.astype(jnp.float32), r1,
                           (((0,), (0,)), ((), ())),
                           preferred_element_type=jnp.float32)
         + be1_ref[...])
    z = jnp.maximum(z, 0.0)
    h = jnp.dot(z, we2_ref[...], preferred_element_type=jnp.float32) + be2_ref[...]
    cs_col = lax.dot_general(cs_ref[...], jnp.ones((1, 1), jnp.float32),
                             (((0,), (0,)), ((), ())),
                             preferred_element_type=jnp.float32)
    hc_ref[...] = h * cs_col

    @pl.when(pl.program_id(0) < _N_NODES // _NODE_BLK)
    def _():
        p = jnp.dot(x_ref[...], wn_ref[...],
                    preferred_element_type=jnp.float32) + bn_ref[...]
        p_ref[...] = jnp.maximum(p, 0.0) * conf_ref[...]


def _tc_fuse(x_ref, a0_ref, a1_ref, wg1_ref, wg2_ref, bg_ref,
             gamma_ref, beta_ref, out_ref):
    x = x_ref[...]
    aggr = a0_ref[...] + a1_ref[...]
    gate = jax.nn.sigmoid(
        jnp.dot(x, wg1_ref[...], preferred_element_type=jnp.float32)
        + jnp.dot(aggr, wg2_ref[...], preferred_element_type=jnp.float32)
        + bg_ref[...])
    fused = gate * jnp.tanh(aggr) + (1.0 - gate) * x
    mean = jnp.mean(fused, axis=-1, keepdims=True)
    ctr = fused - mean
    var = jnp.mean(ctr * ctr, axis=-1, keepdims=True)
    out_ref[...] = ctr * lax.rsqrt(var + 1e-5) * gamma_ref[...] + beta_ref[...]


def kernel(x, edge_index, edge_attr, edge_type, confidence, rel_table,
           We1, be1, We2, be2, Wn, bn, Wg, bg, gamma, beta):
    src = edge_index[0]
    dst = edge_index[1]
    conf1 = confidence.reshape(_N_NODES)

    mesh = plsc.VectorSubcoreMesh(core_axis_name="c", subcore_axis_name="s")

    csrc = pl.kernel(
        _sc_gather_conf_body,
        out_type=jax.ShapeDtypeStruct((_N_EDGES,), jnp.float32),
        mesh=mesh,
        scratch_types=(
            [pltpu.VMEM((_CHUNK,), jnp.int32)] * 4
            + [pltpu.VMEM((_CHUNK,), jnp.float32)] * 4
            + [pltpu.SemaphoreType.DMA] * 4
        ),
    )(src, conf1)

    nblk = _N_NODES // _NODE_BLK
    eblk = _N_EDGES // _EDGE_BLK
    nlast = nblk - 1
    hc, p_nodes = pl.pallas_call(
        _tc_edge_node_mlp,
        grid=(eblk,),
        in_specs=[
            pl.BlockSpec((_EDGE_BLK, _EDGE_DIM), lambda i: (i, 0)),
            pl.BlockSpec((1, _EDGE_BLK), lambda i: (0, i)),
            pl.BlockSpec((1, _EDGE_BLK), lambda i: (0, i)),
            pl.BlockSpec((_NODE_BLK, _H), lambda i: (jnp.minimum(i, nlast), 0)),
            pl.BlockSpec((_NODE_BLK, 1), lambda i: (jnp.minimum(i, nlast), 0)),
            pl.BlockSpec((_NUM_REL, _EDGE_DIM), lambda i: (0, 0)),
            pl.BlockSpec((_EDGE_DIM + _EDGE_DIM, _H), lambda i: (0, 0)),
            pl.BlockSpec((1, _H), lambda i: (0, 0)),
            pl.BlockSpec((_H, _H), lambda i: (0, 0)),
            pl.BlockSpec((1, _H), lambda i: (0, 0)),
            pl.BlockSpec((_H, _H), lambda i: (0, 0)),
            pl.BlockSpec((1, _H), lambda i: (0, 0)),
        ],
        out_specs=[
            pl.BlockSpec((_EDGE_BLK, _H), lambda i: (i, 0)),
            pl.BlockSpec((_NODE_BLK, _H), lambda i: (jnp.minimum(i, nlast), 0)),
        ],
        out_shape=[
            jax.ShapeDtypeStruct((_N_EDGES, _H), jnp.float32),
            jax.ShapeDtypeStruct((_N_NODES, _H), jnp.float32),
        ],
    )(edge_attr, edge_type.reshape(1, _N_EDGES), csrc.reshape(1, _N_EDGES),
      x, confidence, rel_table, We1, be1.reshape(1, _H), We2,
      be2.reshape(1, _H), Wn, bn.reshape(1, _H))

    partials = pl.kernel(
        _sc_scatter_body,
        out_type=jax.ShapeDtypeStruct((_NC * _N_PAD, _H), jnp.float32),
        mesh=mesh,
        scratch_types=[
            pltpu.VMEM((_CHUNK,), jnp.int32),
            pltpu.VMEM((_CHUNK,), jnp.int32),
            pltpu.VMEM((_CHUNK,), jnp.int32),
            pltpu.VMEM((_CHUNK,), jnp.int32),
            pltpu.VMEM((_CHUNK, _H), jnp.float32),
            pltpu.VMEM((_CHUNK, _H), jnp.float32),
            pltpu.VMEM((_CHUNK, _H), jnp.float32),
            pltpu.VMEM((_CHUNK, _H), jnp.float32),
            pltpu.VMEM_SHARED((_N_PAD, _H), jnp.float32),
            pltpu.SemaphoreType.DMA,
            pltpu.SemaphoreType.DMA,
            pltpu.SemaphoreType.DMA,
            pltpu.SemaphoreType.DMA,
        ],
    )(dst, src, hc, p_nodes)

    a0 = partials[:_N_NODES]
    a1 = partials[_N_PAD:_N_PAD + _N_NODES]

    out = pl.pallas_call(
        _tc_fuse,
        grid=(nblk,),
        in_specs=[
            pl.BlockSpec((_NODE_BLK, _H), lambda i: (i, 0)),
            pl.BlockSpec((_NODE_BLK, _H), lambda i: (i, 0)),
            pl.BlockSpec((_NODE_BLK, _H), lambda i: (i, 0)),
            pl.BlockSpec((_H, _H), lambda i: (0, 0)),
            pl.BlockSpec((_H, _H), lambda i: (0, 0)),
            pl.BlockSpec((1, _H), lambda i: (0, 0)),
            pl.BlockSpec((1, _H), lambda i: (0, 0)),
            pl.BlockSpec((1, _H), lambda i: (0, 0)),
        ],
        out_specs=pl.BlockSpec((_NODE_BLK, _H), lambda i: (i, 0)),
        out_shape=jax.ShapeDtypeStruct((_N_NODES, _H), jnp.float32),
    )(x, a0, a1, Wg[:_H], Wg[_H:], bg.reshape(1, _H),
      gamma.reshape(1, _H), beta.reshape(1, _H))
    return out


# speedup vs baseline: 1.7759x; 1.1232x over previous
import functools

import jax
import jax.numpy as jnp
from jax import lax
from jax.experimental import pallas as pl
from jax.experimental.pallas import tpu as pltpu
from jax.experimental.pallas import tpu_sc as plsc

_N_NODES = 10000
_N_EDGES = 320000
_H = 128
_EDGE_DIM = 16
_NUM_REL = 32

_NC = 2
_NS = 16
_NW = _NC * _NS
_EDGES_PER_TILE = _N_EDGES // _NW
_CHUNK = 80
_NCHUNK = _EDGES_PER_TILE // _CHUNK
_N_PAD = 10240
_ROWS_PER_TILE = _N_PAD // _NS
_ZROWS = 80
_NZ = _ROWS_PER_TILE // _ZROWS

_NODE_BLK = 2000
_EDGE_BLK = 2560


_CCHUNK = 128
_CNFULL = _N_EDGES // _CCHUNK // _NW
_CEXTRA = _N_EDGES // _CCHUNK - _CNFULL * _NW


def _sc_gather_conf_body(src_hbm, conf_hbm, out_hbm,
                         idx0, idx1, idx2, idx3, val0, val1, val2, val3,
                         sg0, sg1, sg2, sg3):
    c = lax.axis_index("c")
    s = lax.axis_index("s")
    wid = s * _NC + c
    base0 = wid * _CNFULL * _CCHUNK
    idx = (idx0, idx1, idx2, idx3)
    val = (val0, val1, val2, val3)
    sg = (sg0, sg1, sg2, sg3)

    def stage_load(g, b):
        base = base0 + g * _CCHUNK
        pltpu.sync_copy(src_hbm.at[pl.ds(base, _CCHUNK)], idx[b])
        pltpu.async_copy(conf_hbm.at[idx[b]], val[b], sg[b])

    def stage_drain(g, b):
        pltpu.make_async_copy(conf_hbm.at[idx[b]], val[b], sg[b]).wait()
        base = base0 + g * _CCHUNK
        pltpu.sync_copy(val[b], out_hbm.at[pl.ds(base, _CCHUNK)])

    for g in range(3):
        stage_load(g, g)

    def body(k, carry):
        for r in range(4):
            g = 4 * k + r
            stage_drain(g, r)

            @pl.when(g + 3 < _CNFULL)
            def _():
                stage_load(g + 3, (r + 3) % 4)
        return carry

    lax.fori_loop(0, _CNFULL // 4, body, 0)
    for g in range(4 * (_CNFULL // 4), _CNFULL):
        stage_drain(g, g % 4)

    @pl.when(wid < _CEXTRA)
    def _():
        base = (_NW * _CNFULL + wid) * _CCHUNK
        pltpu.sync_copy(src_hbm.at[pl.ds(base, _CCHUNK)], idx0)
        pltpu.sync_copy(conf_hbm.at[idx0], val0)
        pltpu.sync_copy(val0, out_hbm.at[pl.ds(base, _CCHUNK)])


def _sc_scatter_body(dst_hbm, src_hbm, hc_hbm, p_hbm, out_hbm,
                     dstv0, dstv1, srcv0, srcv1, hbuf0, hbuf1, pbuf0, pbuf1,
                     aggr_sh, sl0, sl1, sg0, sg1):
    c = lax.axis_index("c")
    s = lax.axis_index("s")
    wid = s * _NC + c
    base0 = wid * _EDGES_PER_TILE
    dstv = (dstv0, dstv1)
    srcv = (srcv0, srcv1)
    hbuf = (hbuf0, hbuf1)
    pbuf = (pbuf0, pbuf1)
    sl = (sl0, sl1)
    sg = (sg0, sg1)

    def issue_loads(g, b):
        base = base0 + g * _CHUNK
        pltpu.async_copy(dst_hbm.at[pl.ds(base, _CHUNK)], dstv[b], sl[b])
        pltpu.async_copy(src_hbm.at[pl.ds(base, _CHUNK)], srcv[b], sl[b])
        pltpu.async_copy(hc_hbm.at[pl.ds(base, _CHUNK)], hbuf[b], sl[b])

    def wait_loads(g, b):
        base = base0 + g * _CHUNK
        pltpu.make_async_copy(dst_hbm.at[pl.ds(base, _CHUNK)], dstv[b], sl[b]).wait()
        pltpu.make_async_copy(src_hbm.at[pl.ds(base, _CHUNK)], srcv[b], sl[b]).wait()
        pltpu.make_async_copy(hc_hbm.at[pl.ds(base, _CHUNK)], hbuf[b], sl[b]).wait()

    def issue_gather(b):
        pltpu.async_copy(p_hbm.at[srcv[b]], pbuf[b], sg[b])

    def wait_gather(b):
        pltpu.make_async_copy(p_hbm.at[srcv[b]], pbuf[b], sg[b]).wait()

    def scatter(b):
        pltpu.sync_copy(hbuf[b], aggr_sh.at[dstv[b]], add=True)
        pltpu.sync_copy(pbuf[b], aggr_sh.at[dstv[b]], add=True)

    issue_loads(0, 0)
    issue_loads(1, 1)

    z16 = jnp.zeros((16,), jnp.float32)

    def zrow(i, carry):
        for j in range(_H // 16):
            pbuf0[i, pl.ds(j * 16, 16)] = z16
        return carry

    lax.fori_loop(0, _ZROWS, zrow, 0)
    for k in range(_NZ):
        pltpu.sync_copy(
            pbuf0, aggr_sh.at[pl.ds(s * _ROWS_PER_TILE + k * _ZROWS, _ZROWS)])
    plsc.subcore_barrier()

    wait_loads(0, 0)
    issue_gather(0)

    def step(g, b):
        wait_loads(g + 1, 1 - b)
        issue_gather(1 - b)
        wait_gather(b)
        scatter(b)

        @pl.when(g + 2 < _NCHUNK)
        def _():
            issue_loads(g + 2, b)

    def body(k, carry):
        step(2 * k, 0)
        step(2 * k + 1, 1)
        return carry

    lax.fori_loop(0, _NCHUNK // 2, body, 0)
    wait_gather(0)
    scatter(0)
    plsc.subcore_barrier()

    for k in range(_NZ):
        r = s * _ROWS_PER_TILE + k * _ZROWS
        pltpu.sync_copy(aggr_sh.at[pl.ds(r, _ZROWS)],
                        out_hbm.at[pl.ds(c * _N_PAD + r, _ZROWS)])


def _tc_edge_node_mlp(eat_ref, et_ref, cs_ref, x_ref, conf_ref, rel_ref,
                      we1_ref, be1_ref, we2_ref, be2_ref, wn_ref, bn_ref,
                      hc_ref, p_ref):
    we1a = we1_ref[:_EDGE_DIM, :]
    we1b = we1_ref[_EDGE_DIM:, :]
    r1 = jnp.dot(rel_ref[...], we1b, preferred_element_type=jnp.float32)
    oh_t = (et_ref[...] == lax.broadcasted_iota(jnp.int32, (_NUM_REL, 1), 0))
    z = (lax.dot_general(eat_ref[...], we1a, (((0,), (0,)), ((), ())),
                         preferred_element_type=jnp.float32)
         + lax.dot_general(oh_t.astype(jnp.float32), r1,
                           (((0,), (0,)), ((), ())),
                           preferred_element_type=jnp.float32)
         + be1_ref[...])
    z = jnp.maximum(z, 0.0)
    h = jnp.dot(z, we2_ref[...], preferred_element_type=jnp.float32) + be2_ref[...]
    cs_col = lax.dot_general(cs_ref[...], jnp.ones((1, 1), jnp.float32),
                             (((0,), (0,)), ((), ())),
                             preferred_element_type=jnp.float32)
    hc_ref[...] = h * cs_col

    @pl.when(pl.program_id(0) < _N_NODES // _NODE_BLK)
    def _():
        p = jnp.dot(x_ref[...], wn_ref[...],
                    preferred_element_type=jnp.float32) + bn_ref[...]
        p_ref[...] = jnp.maximum(p, 0.0) * conf_ref[...]


def _tc_fuse(x_ref, a0_ref, a1_ref, wg1_ref, wg2_ref, bg_ref,
             gamma_ref, beta_ref, out_ref):
    x = x_ref[...]
    aggr = a0_ref[...] + a1_ref[...]
    gate = jax.nn.sigmoid(
        jnp.dot(x, wg1_ref[...], preferred_element_type=jnp.float32)
        + jnp.dot(aggr, wg2_ref[...], preferred_element_type=jnp.float32)
        + bg_ref[...])
    fused = gate * jnp.tanh(aggr) + (1.0 - gate) * x
    mean = jnp.mean(fused, axis=-1, keepdims=True)
    ctr = fused - mean
    var = jnp.mean(ctr * ctr, axis=-1, keepdims=True)
    out_ref[...] = ctr * lax.rsqrt(var + 1e-5) * gamma_ref[...] + beta_ref[...]


def kernel(x, edge_index, edge_attr, edge_type, confidence, rel_table,
           We1, be1, We2, be2, Wn, bn, Wg, bg, gamma, beta):
    src = edge_index[0]
    dst = edge_index[1]
    conf1 = confidence.reshape(_N_NODES)

    mesh = plsc.VectorSubcoreMesh(core_axis_name="c", subcore_axis_name="s")

    csrc = pl.kernel(
        _sc_gather_conf_body,
        out_type=jax.ShapeDtypeStruct((_N_EDGES,), jnp.float32),
        mesh=mesh,
        scratch_types=(
            [pltpu.VMEM((_CCHUNK,), jnp.int32)] * 4
            + [pltpu.VMEM((_CCHUNK,), jnp.float32)] * 4
            + [pltpu.SemaphoreType.DMA] * 4
        ),
    )(src, conf1)

    nblk = _N_NODES // _NODE_BLK
    eblk = _N_EDGES // _EDGE_BLK
    nlast = nblk - 1
    hc, p_nodes = pl.pallas_call(
        _tc_edge_node_mlp,
        grid=(eblk,),
        in_specs=[
            pl.BlockSpec((_EDGE_DIM, _EDGE_BLK), lambda i: (0, i)),
            pl.BlockSpec((1, _EDGE_BLK), lambda i: (0, i)),
            pl.BlockSpec((1, _EDGE_BLK), lambda i: (0, i)),
            pl.BlockSpec((_NODE_BLK, _H), lambda i: (jnp.minimum(i, nlast), 0)),
            pl.BlockSpec((_NODE_BLK, 1), lambda i: (jnp.minimum(i, nlast), 0)),
            pl.BlockSpec((_NUM_REL, _EDGE_DIM), lambda i: (0, 0)),
            pl.BlockSpec((_EDGE_DIM + _EDGE_DIM, _H), lambda i: (0, 0)),
            pl.BlockSpec((1, _H), lambda i: (0, 0)),
            pl.BlockSpec((_H, _H), lambda i: (0, 0)),
            pl.BlockSpec((1, _H), lambda i: (0, 0)),
            pl.BlockSpec((_H, _H), lambda i: (0, 0)),
            pl.BlockSpec((1, _H), lambda i: (0, 0)),
        ],
        out_specs=[
            pl.BlockSpec((_EDGE_BLK, _H), lambda i: (i, 0)),
            pl.BlockSpec((_NODE_BLK, _H), lambda i: (jnp.minimum(i, nlast), 0)),
        ],
        out_shape=[
            jax.ShapeDtypeStruct((_N_EDGES, _H), jnp.float32),
            jax.ShapeDtypeStruct((_N_NODES, _H), jnp.float32),
        ],
    )(edge_attr.T, edge_type.reshape(1, _N_EDGES), csrc.reshape(1, _N_EDGES),
      x, confidence, rel_table, We1, be1.reshape(1, _H), We2,
      be2.reshape(1, _H), Wn, bn.reshape(1, _H))

    partials = pl.kernel(
        _sc_scatter_body,
        out_type=jax.ShapeDtypeStruct((_NC * _N_PAD, _H), jnp.float32),
        mesh=mesh,
        scratch_types=[
            pltpu.VMEM((_CHUNK,), jnp.int32),
            pltpu.VMEM((_CHUNK,), jnp.int32),
            pltpu.VMEM((_CHUNK,), jnp.int32),
            pltpu.VMEM((_CHUNK,), jnp.int32),
            pltpu.VMEM((_CHUNK, _H), jnp.float32),
            pltpu.VMEM((_CHUNK, _H), jnp.float32),
            pltpu.VMEM((_CHUNK, _H), jnp.float32),
            pltpu.VMEM((_CHUNK, _H), jnp.float32),
            pltpu.VMEM_SHARED((_N_PAD, _H), jnp.float32),
            pltpu.SemaphoreType.DMA,
            pltpu.SemaphoreType.DMA,
            pltpu.SemaphoreType.DMA,
            pltpu.SemaphoreType.DMA,
        ],
    )(dst, src, hc, p_nodes)

    a0 = partials[:_N_NODES]
    a1 = partials[_N_PAD:_N_PAD + _N_NODES]

    out = pl.pallas_call(
        _tc_fuse,
        grid=(nblk,),
        in_specs=[
            pl.BlockSpec((_NODE_BLK, _H), lambda i: (i, 0)),
            pl.BlockSpec((_NODE_BLK, _H), lambda i: (i, 0)),
            pl.BlockSpec((_NODE_BLK, _H), lambda i: (i, 0)),
            pl.BlockSpec((_H, _H), lambda i: (0, 0)),
            pl.BlockSpec((_H, _H), lambda i: (0, 0)),
            pl.BlockSpec((1, _H), lambda i: (0, 0)),
            pl.BlockSpec((1, _H), lambda i: (0, 0)),
            pl.BlockSpec((1, _H), lambda i: (0, 0)),
        ],
        out_specs=pl.BlockSpec((_NODE_BLK, _H), lambda i: (i, 0)),
        out_shape=jax.ShapeDtypeStruct((_N_NODES, _H), jnp.float32),
    )(x, a0, a1, Wg[:_H], Wg[_H:], bg.reshape(1, _H),
      gamma.reshape(1, _H), beta.reshape(1, _H))
    return out
